# baseline (speedup 1.0000x reference)
import functools
import math

import jax
import jax.numpy as jnp
from jax import lax
from jax.experimental import pallas as pl
from jax.experimental.pallas import tpu as pltpu
from jax.experimental.pallas import tpu_sc as plsc

_B = 4096
_L = 200
_EMB = 64
_NC = 2
_NS = 16
_NW = _NC * _NS
_SEQ_PER_W = _B // _NW
_L0 = 128
_L1 = _L - _L0
_NLANE = 16
_NVEC = _EMB // _NLANE


def _gather_sum_body(tok_hbm, table_hbm, emb_hbm, sums_hbm,
                     tok_v, rows_v, sums_v, sem):
    wid = lax.axis_index("s") * _NC + lax.axis_index("c")
    seq0 = wid * _SEQ_PER_W
    pltpu.sync_copy(tok_hbm.at[pl.ds(seq0, _SEQ_PER_W)], tok_v)

    def seq_body(b, carry):
        g0 = pltpu.async_copy(
            table_hbm.at[tok_v.at[b, pl.ds(0, _L0)]],
            rows_v.at[pl.ds(0, _L0)], sem)
        g1 = pltpu.async_copy(
            table_hbm.at[tok_v.at[b, pl.ds(_L0, _L1)]],
            rows_v.at[pl.ds(_L0, _L1)], sem)
        g0.wait()
        g1.wait()
        pltpu.sync_copy(rows_v, emb_hbm.at[pl.ds((seq0 + b) * _L, _L)])

        def acc_body(t, accs):
            return tuple(accs[j] + rows_v[t, pl.ds(_NLANE * j, _NLANE)]
                         for j in range(_NVEC))
        z = jnp.zeros((_NLANE,), jnp.float32)
        accs = lax.fori_loop(0, _L, acc_body, (z,) * _NVEC)
        for j in range(_NVEC):
            sums_v[b, pl.ds(_NLANE * j, _NLANE)] = accs[j]
        return carry

    lax.fori_loop(0, _SEQ_PER_W, seq_body, 0)
    pltpu.sync_copy(sums_v, sums_hbm.at[pl.ds(seq0, _SEQ_PER_W)])


_gather_sum = functools.partial(
    pl.kernel,
    mesh=plsc.VectorSubcoreMesh(core_axis_name="c", subcore_axis_name="s"),
    compiler_params=pltpu.CompilerParams(use_tc_tiling_on_sc=False),
    out_type=(
        jax.ShapeDtypeStruct((_B * _L, _EMB), jnp.float32),
        jax.ShapeDtypeStruct((_B, _EMB), jnp.float32),
    ),
    scratch_types=(
        pltpu.VMEM((_SEQ_PER_W, _L), jnp.int32),
        pltpu.VMEM((_L, _EMB), jnp.float32),
        pltpu.VMEM((_SEQ_PER_W, _EMB), jnp.float32),
        pltpu.SemaphoreType.DMA,
    ),
)(_gather_sum_body)


def _gelu(x):
    return 0.5 * x * (1.0 + lax.erf(x * (1.0 / math.sqrt(2.0))))


def _mlp_body(s_ref, w1_ref, b1_ref, w2_ref, b2_ref, w3_ref, b3_ref, o_ref):
    x = s_ref[...] * (1.0 / _L)
    h = lax.dot_general(x, w1_ref[...], (((1,), (1,)), ((), ())),
                        preferred_element_type=jnp.float32)
    h = _gelu(h + b1_ref[...])
    h = lax.dot_general(h, w2_ref[...], (((1,), (1,)), ((), ())),
                        preferred_element_type=jnp.float32)
    h = _gelu(h + b2_ref[...])
    e = lax.dot_general(w3_ref[...], h, (((1,), (1,)), ((), ())),
                        preferred_element_type=jnp.float32)
    o_ref[...] = e + b3_ref[0]


def kernel(token_ids, table, W1, b1, W2, b2, W3, b3):
    emb_flat, sums = _gather_sum(token_ids.astype(jnp.int32), table)
    energy_row = pl.pallas_call(
        _mlp_body,
        in_specs=[
            pl.BlockSpec(memory_space=pltpu.VMEM),
            pl.BlockSpec(memory_space=pltpu.VMEM),
            pl.BlockSpec(memory_space=pltpu.VMEM),
            pl.BlockSpec(memory_space=pltpu.VMEM),
            pl.BlockSpec(memory_space=pltpu.VMEM),
            pl.BlockSpec(memory_space=pltpu.VMEM),
            pl.BlockSpec(memory_space=pltpu.SMEM),
        ],
        out_specs=pl.BlockSpec(memory_space=pltpu.VMEM),
        out_shape=jax.ShapeDtypeStruct((1, _B), jnp.float32),
    )(sums, W1, b1.reshape(1, -1), W2, b2.reshape(1, -1), W3, b3)
    return energy_row.reshape(_B, 1), emb_flat.reshape(_B, _L, _EMB)


# speedup vs baseline: 1.1385x; 1.1385x over previous
import functools
import math

import jax
import jax.numpy as jnp
from jax import lax
from jax.experimental import pallas as pl
from jax.experimental.pallas import tpu as pltpu
from jax.experimental.pallas import tpu_sc as plsc

_B = 4096
_L = 200
_EMB = 64
_NC = 2
_NS = 16
_NW = _NC * _NS
_SEQ_PER_W = _B // _NW
_L0 = 128
_L1 = _L - _L0
_NLANE = 16
_NVEC = _EMB // _NLANE


_NBUF = 4
_TUNROLL = 8


def _gather_sum_body(tok_hbm, table_hbm, emb_hbm, sums_hbm,
                     tok_v, rows_v, sums_v,
                     gs0, gs1, gs2, gs3, ws0, ws1, ws2, ws3):
    gsems = (gs0, gs1, gs2, gs3)
    wsems = (ws0, ws1, ws2, ws3)
    wid = lax.axis_index("s") * _NC + lax.axis_index("c")
    seq0 = wid * _SEQ_PER_W
    pltpu.sync_copy(tok_hbm.at[pl.ds(seq0, _SEQ_PER_W)], tok_v)

    def issue_gather(b, k):
        pltpu.async_copy(
            table_hbm.at[tok_v.at[b, pl.ds(0, _L0)]],
            rows_v.at[k, pl.ds(0, _L0)], gsems[k])
        pltpu.async_copy(
            table_hbm.at[tok_v.at[b, pl.ds(_L0, _L1)]],
            rows_v.at[k, pl.ds(_L0, _L1)], gsems[k])

    def wait_bytes(k, sem):
        pltpu.make_async_copy(rows_v.at[k], emb_hbm.at[pl.ds(0, _L)],
                              sem).wait()

    for k in range(_NBUF - 1):
        issue_gather(k, k)

    def group_body(g, carry):
        for k in range(_NBUF):
            b = g * _NBUF + k
            wait_bytes(k, gsems[k])
            pltpu.async_copy(rows_v.at[k],
                             emb_hbm.at[pl.ds((seq0 + b) * _L, _L)],
                             wsems[k])
            def acc_body(tt, accs, k=k):
                new = list(accs)
                for i in range(_TUNROLL):
                    t = tt * _TUNROLL + i
                    for j in range(_NVEC):
                        new[j] = new[j] + rows_v[k, t, pl.ds(_NLANE * j,
                                                             _NLANE)]
                return tuple(new)
            z = jnp.zeros((_NLANE,), jnp.float32)
            accs = lax.fori_loop(0, _L // _TUNROLL, acc_body, (z,) * _NVEC)
            for j in range(_NVEC):
                sums_v[b, pl.ds(_NLANE * j, _NLANE)] = accs[j]
            kn = (k + _NBUF - 1) % _NBUF
            @pl.when(b == 0)
            def _():
                issue_gather(b + _NBUF - 1, kn)
            @pl.when((b >= 1) & (b <= _SEQ_PER_W - _NBUF))
            def _():
                wait_bytes(kn, wsems[kn])
                issue_gather(b + _NBUF - 1, kn)
        return carry

    lax.fori_loop(0, _SEQ_PER_W // _NBUF, group_body, 0)
    for k in range(_NBUF):
        wait_bytes(k, wsems[k])
    pltpu.sync_copy(sums_v, sums_hbm.at[pl.ds(seq0, _SEQ_PER_W)])


_gather_sum = functools.partial(
    pl.kernel,
    mesh=plsc.VectorSubcoreMesh(core_axis_name="c", subcore_axis_name="s"),
    compiler_params=pltpu.CompilerParams(use_tc_tiling_on_sc=False),
    out_type=(
        jax.ShapeDtypeStruct((_B * _L, _EMB), jnp.float32),
        jax.ShapeDtypeStruct((_B, _EMB), jnp.float32),
    ),
    scratch_types=(
        pltpu.VMEM((_SEQ_PER_W, _L), jnp.int32),
        pltpu.VMEM((_NBUF, _L, _EMB), jnp.float32),
        pltpu.VMEM((_SEQ_PER_W, _EMB), jnp.float32),
        pltpu.SemaphoreType.DMA,
        pltpu.SemaphoreType.DMA,
        pltpu.SemaphoreType.DMA,
        pltpu.SemaphoreType.DMA,
        pltpu.SemaphoreType.DMA,
        pltpu.SemaphoreType.DMA,
        pltpu.SemaphoreType.DMA,
        pltpu.SemaphoreType.DMA,
    ),
)(_gather_sum_body)


def _gelu(x):
    return 0.5 * x * (1.0 + lax.erf(x * (1.0 / math.sqrt(2.0))))


def _mlp_body(s_ref, w1_ref, b1_ref, w2_ref, b2_ref, w3_ref, b3_ref, o_ref):
    x = s_ref[...] * (1.0 / _L)
    h = lax.dot_general(x, w1_ref[...], (((1,), (1,)), ((), ())),
                        preferred_element_type=jnp.float32)
    h = _gelu(h + b1_ref[...])
    h = lax.dot_general(h, w2_ref[...], (((1,), (1,)), ((), ())),
                        preferred_element_type=jnp.float32)
    h = _gelu(h + b2_ref[...])
    e = lax.dot_general(w3_ref[...], h, (((1,), (1,)), ((), ())),
                        preferred_element_type=jnp.float32)
    o_ref[...] = e + b3_ref[0]


def kernel(token_ids, table, W1, b1, W2, b2, W3, b3):
    emb_flat, sums = _gather_sum(token_ids.astype(jnp.int32), table)
    energy_row = pl.pallas_call(
        _mlp_body,
        in_specs=[
            pl.BlockSpec(memory_space=pltpu.VMEM),
            pl.BlockSpec(memory_space=pltpu.VMEM),
            pl.BlockSpec(memory_space=pltpu.VMEM),
            pl.BlockSpec(memory_space=pltpu.VMEM),
            pl.BlockSpec(memory_space=pltpu.VMEM),
            pl.BlockSpec(memory_space=pltpu.VMEM),
            pl.BlockSpec(memory_space=pltpu.SMEM),
        ],
        out_specs=pl.BlockSpec(memory_space=pltpu.VMEM),
        out_shape=jax.ShapeDtypeStruct((1, _B), jnp.float32),
    )(sums, W1, b1.reshape(1, -1), W2, b2.reshape(1, -1), W3, b3)
    return energy_row.reshape(_B, 1), emb_flat.reshape(_B, _L, _EMB)
